# Initial kernel scaffold; baseline (speedup 1.0000x reference)
import jax
import jax.numpy as jnp
from jax.experimental import pallas as pl


def kernel(x, sW1, sb1, sW2, sb2, spWr, spbr, spW1, spb1, spW2, spb2, scWr, scbr, scW1, scb1, scW2, scb2):
    raise NotImplementedError("write your pallas kernel here")


# speedup vs baseline: 1.0982x; 1.0982x over previous
import functools
import math

import jax
import jax.numpy as jnp
from jax import lax
from jax.experimental import pallas as pl
from jax.experimental.pallas import tpu as pltpu
from jax.experimental.pallas import tpu_sc as plsc

_B = 1
_S = 2048
_D = 1024
_F = 4096
_E = 8
_T = _B * _S
_CAP = int(math.ceil(1.0 * _T / _E))
_NW = 32
_CH = _T // _NW


def _router_body(xf_ref, wr_sp_ref, br_sp_ref, wr_sc_ref, br_sc_ref,
                 slot_sp_ref, sg_sp_ref, w_sp_ref,
                 slot_sc_ref, sg_sc_ref, w_sc_ref):
    xf = xf_ref[...]

    def bank(wr_ref, br_ref, slot_ref, sg_ref, w_ref):
        lg = lax.dot_general(wr_ref[...], xf, (((0,), (1,)), ((), ())))
        lg = lg + br_ref[...]
        m = jnp.max(lg, axis=0, keepdims=True)
        p = jnp.exp(lg - m)
        g = 1.0 / jnp.sum(p, axis=0, keepdims=True)
        taken = jnp.zeros((1, _T), dtype=jnp.bool_)
        rows = []
        for i in range(_E):
            eqi = lg[i:i + 1, :] == m
            rows.append(jnp.where(jnp.logical_and(eqi, jnp.logical_not(taken)),
                                  1.0, 0.0))
            taken = jnp.logical_or(taken, eqi)
        oh = jnp.concatenate(rows, axis=0)
        eidx = jnp.sum(
            oh * lax.broadcasted_iota(jnp.int32, (_E, _T), 0).astype(jnp.float32),
            axis=0, keepdims=True)
        ps = oh
        sh = 1
        while sh < _T:
            ps = ps + jnp.concatenate(
                [jnp.zeros((_E, sh), jnp.float32), ps[:, :_T - sh]], axis=1)
            sh *= 2
        pos = jnp.sum(ps * oh, axis=0, keepdims=True)
        keep = pos <= float(_CAP)
        slot_f = eidx * float(_CAP) + pos - 1.0
        slot_ref[...] = jnp.where(keep, slot_f,
                                  float(_E * _CAP)).astype(jnp.int32)
        sg_ref[...] = jnp.where(keep, slot_f,
                                (eidx + 1.0) * float(_CAP) - 1.0).astype(jnp.int32)
        w_ref[...] = jnp.where(keep, g, 0.0)

    bank(wr_sp_ref, br_sp_ref, slot_sp_ref, sg_sp_ref, w_sp_ref)
    bank(wr_sc_ref, br_sc_ref, slot_sc_ref, sg_sc_ref, w_sc_ref)


def _route(xf, spWr, spbr, scWr, scbr):
    i32 = jax.ShapeDtypeStruct((1, _T), jnp.int32)
    f32 = jax.ShapeDtypeStruct((1, _T), jnp.float32)
    return pl.pallas_call(
        _router_body,
        out_shape=(i32, i32, f32, i32, i32, f32),
    )(xf, spWr, spbr.reshape(_E, 1), scWr, scbr.reshape(_E, 1))


def _worker_base():
    wid = lax.axis_index("s") * 2 + lax.axis_index("c")
    return wid * _CH


@functools.cache
def _sc_kernels():
    mesh = plsc.VectorSubcoreMesh(core_axis_name="c", subcore_axis_name="s")

    @functools.partial(
        pl.kernel,
        out_type=(jax.ShapeDtypeStruct((_T + 1, _D), jnp.float32),
                  jax.ShapeDtypeStruct((_T + 1, _D), jnp.float32)),
        mesh=mesh,
        scratch_types=[pltpu.VMEM((_CH,), jnp.int32),
                       pltpu.VMEM((_CH, _D), jnp.float32),
                       pltpu.SemaphoreType.DMA],
    )
    def _sc_scatter(xf_hbm, slot_sp_hbm, slot_sc_hbm, ein_sp_hbm, ein_sc_hbm,
                    idx_v, rows_v, sem):
        base = _worker_base()
        pltpu.sync_copy(xf_hbm.at[pl.ds(base, _CH)], rows_v)
        pltpu.sync_copy(slot_sp_hbm.at[pl.ds(base, _CH)], idx_v)
        pltpu.async_copy(rows_v, ein_sp_hbm.at[idx_v], sem).wait()
        pltpu.sync_copy(slot_sc_hbm.at[pl.ds(base, _CH)], idx_v)
        pltpu.async_copy(rows_v, ein_sc_hbm.at[idx_v], sem).wait()

    @functools.partial(
        pl.kernel,
        out_type=(jax.ShapeDtypeStruct((_T, _D), jnp.float32),
                  jax.ShapeDtypeStruct((_T, _D), jnp.float32)),
        mesh=mesh,
        scratch_types=[pltpu.VMEM((_CH,), jnp.int32),
                       pltpu.VMEM((_CH, _D), jnp.float32),
                       pltpu.SemaphoreType.DMA],
    )
    def _sc_gather(out_sp_hbm, out_sc_hbm, sg_sp_hbm, sg_sc_hbm,
                   y_sp_hbm, y_sc_hbm, idx_v, rows_v, sem):
        base = _worker_base()
        pltpu.sync_copy(sg_sp_hbm.at[pl.ds(base, _CH)], idx_v)
        pltpu.async_copy(out_sp_hbm.at[idx_v], rows_v, sem).wait()
        pltpu.sync_copy(rows_v, y_sp_hbm.at[pl.ds(base, _CH)])
        pltpu.sync_copy(sg_sc_hbm.at[pl.ds(base, _CH)], idx_v)
        pltpu.async_copy(out_sc_hbm.at[idx_v], rows_v, sem).wait()
        pltpu.sync_copy(rows_v, y_sc_hbm.at[pl.ds(base, _CH)])

    return _sc_scatter, _sc_gather


_FH = _F // 2


def _ffn_body(x_ref, w1_ref, b1_ref, w2_ref, b2_ref, o_ref):
    f = pl.program_id(1)
    xb = x_ref[...].astype(jnp.bfloat16)
    w1 = w1_ref[0].astype(jnp.bfloat16)
    h = jnp.dot(xb, w1, preferred_element_type=jnp.float32) + b1_ref[0]
    h = jax.nn.gelu(h)
    part = jnp.dot(h.astype(jnp.bfloat16), w2_ref[0].astype(jnp.bfloat16),
                   preferred_element_type=jnp.float32)

    @pl.when(f == 0)
    def _():
        o_ref[...] = part + b2_ref[0]

    @pl.when(f != 0)
    def _():
        o_ref[...] += part


def _expert_ffn(ein, W1, b1, W2, b2):
    return pl.pallas_call(
        _ffn_body,
        grid=(_E, _F // _FH),
        in_specs=[
            pl.BlockSpec((_CAP, _D), lambda e, f: (e, 0)),
            pl.BlockSpec((1, _D, _FH), lambda e, f: (e, 0, f)),
            pl.BlockSpec((1, 1, _FH), lambda e, f: (e, 0, f)),
            pl.BlockSpec((1, _FH, _D), lambda e, f: (e, f, 0)),
            pl.BlockSpec((1, 1, _D), lambda e, f: (e, 0, 0)),
        ],
        out_specs=pl.BlockSpec((_CAP, _D), lambda e, f: (e, 0)),
        out_shape=jax.ShapeDtypeStruct((_T, _D), jnp.float32),
    )(ein, W1, b1.reshape(_E, 1, _F), W2, b2.reshape(_E, 1, _D))


def _shared_body(x_ref, w1_ref, b1_ref, w2_ref, b2_ref,
                 ysp_ref, ysc_ref, wsp_ref, wsc_ref, o_ref):
    xb = x_ref[...].astype(jnp.bfloat16)
    h = jnp.dot(xb, w1_ref[...], preferred_element_type=jnp.float32) + b1_ref[...]
    h = jax.nn.gelu(h)
    out = jnp.dot(h.astype(jnp.bfloat16), w2_ref[...],
                  preferred_element_type=jnp.float32) + b2_ref[...]
    o_ref[...] = out + ysp_ref[...] * wsp_ref[...] + ysc_ref[...] * wsc_ref[...]


def _shared_combine(xf, sW1, sb1, sW2, sb2, ysp, ysc, wsp, wsc):
    blk = _T // 8
    return pl.pallas_call(
        _shared_body,
        grid=(8,),
        in_specs=[
            pl.BlockSpec((blk, _D), lambda i: (i, 0)),
            pl.BlockSpec((_D, _F), lambda i: (0, 0)),
            pl.BlockSpec((1, _F), lambda i: (0, 0)),
            pl.BlockSpec((_F, _D), lambda i: (0, 0)),
            pl.BlockSpec((1, _D), lambda i: (0, 0)),
            pl.BlockSpec((blk, _D), lambda i: (i, 0)),
            pl.BlockSpec((blk, _D), lambda i: (i, 0)),
            pl.BlockSpec((blk, 1), lambda i: (i, 0)),
            pl.BlockSpec((blk, 1), lambda i: (i, 0)),
        ],
        out_specs=pl.BlockSpec((blk, _D), lambda i: (i, 0)),
        out_shape=jax.ShapeDtypeStruct((_T, _D), jnp.float32),
    )(xf, sW1.astype(jnp.bfloat16), sb1.reshape(1, _F),
      sW2.astype(jnp.bfloat16), sb2.reshape(1, _D),
      ysp, ysc, wsp, wsc)


def kernel(x, sW1, sb1, sW2, sb2, spWr, spbr, spW1, spb1, spW2, spb2,
           scWr, scbr, scW1, scb1, scW2, scb2):
    xf = x.reshape(_T, _D)
    slot_sp, sg_sp, w_sp, slot_sc, sg_sc, w_sc = _route(
        xf, spWr, spbr, scWr, scbr)
    sc_scatter, sc_gather = _sc_kernels()
    ein_sp_full, ein_sc_full = sc_scatter(
        xf, slot_sp.reshape(_T), slot_sc.reshape(_T))
    out_sp = _expert_ffn(ein_sp_full[:_T], spW1, spb1, spW2, spb2)
    out_sc = _expert_ffn(ein_sc_full[:_T], scW1, scb1, scW2, scb2)
    y_sp, y_sc = sc_gather(out_sp, out_sc,
                           sg_sp.reshape(_T), sg_sc.reshape(_T))
    y = _shared_combine(xf, sW1, sb1, sW2, sb2, y_sp, y_sc,
                        w_sp.reshape(_T, 1), w_sc.reshape(_T, 1))
    return y.reshape(_B, _S, _D)
